# Initial kernel scaffold; baseline (speedup 1.0000x reference)
import jax
import jax.numpy as jnp
from jax.experimental import pallas as pl


def kernel(x, mask):
    raise NotImplementedError("write your pallas kernel here")


# speedup vs baseline: 41.6060x; 41.6060x over previous
import jax
import jax.numpy as jnp
from jax.experimental import pallas as pl
from jax.experimental.pallas import tpu as pltpu

_SPARSITY = 0.5


def _select_mask_kernel(k, x_ref, o_ref):
    x = x_ref[...]
    u = jax.lax.bitcast_convert_type(x, jnp.int32) & jnp.int32(0x7FFFFFFF)

    def body(i, lo):
        cand = lo | (jnp.int32(1) << (jnp.int32(30) - i))
        cnt = jnp.sum((u < cand).astype(jnp.int32))
        return jnp.where(cnt >= jnp.int32(k), lo, cand)

    t = jax.lax.fori_loop(0, 31, body, jnp.int32(0))
    o_ref[...] = jnp.where(u >= t, x, jnp.float32(0.0))


def kernel(x, mask):
    del mask
    n = x.size
    idx = max(int(_SPARSITY * n - 1), 0)
    k = idx + 1
    import functools

    return pl.pallas_call(
        functools.partial(_select_mask_kernel, k),
        out_shape=jax.ShapeDtypeStruct(x.shape, x.dtype),
    )(x)
